# Initial kernel scaffold; baseline (speedup 1.0000x reference)
import jax
import jax.numpy as jnp
from jax.experimental import pallas as pl


def kernel(edge_index, emb, W1, b1, W2, b2, Wh, bh):
    raise NotImplementedError("write your pallas kernel here")


# speedup vs baseline: 11.1468x; 11.1468x over previous
import functools

import jax
import jax.numpy as jnp
from jax import lax
from jax.experimental import pallas as pl
from jax.experimental.pallas import tpu as pltpu
from jax.experimental.pallas import tpu_sc as plsc

N_NODES = 100_000
DIM = 32
N_EDGES = 1_600_000

NC = 2
NS = 16
LANES = 16

ROWS = 12_800
EP = ROWS * 128

DEG_WORDS = 100_352
DEG_TILE = DEG_WORDS // NS
DEG_ROWS_PER_SC = ROWS // NC
DEG_ROWS_PER_TILE = DEG_ROWS_PER_SC // NS
DEG_CHUNKS = DEG_ROWS_PER_TILE // 8

LOCAL = N_NODES // NC
ACC_ROWS = 51_200
DUMMY = LOCAL
ZCHUNKS = ACC_ROWS // NS // 128
ROWS_PER_TILE = ROWS // NS
CROWS = 4
CHUNKS = ROWS_PER_TILE // CROWS
CP_ROWS = 400
CP_CHUNKS = LOCAL // CP_ROWS


def _deg_body(dst2d, deg_out, zbuf, ones_v, stage, deg_sh):
    cid = lax.axis_index("c")
    tid = lax.axis_index("s")
    zeros16 = jnp.zeros((LANES,), jnp.float32)
    ones16 = jnp.ones((LANES,), jnp.float32)

    def zb(i, _):
        zbuf[pl.ds(i * LANES, LANES)] = zeros16
        return 0
    lax.fori_loop(0, DEG_TILE // LANES, zb, 0)
    for i in range(128 // LANES):
        ones_v[pl.ds(i * LANES, LANES)] = ones16

    pltpu.sync_copy(zbuf, deg_sh.at[pl.ds(tid * DEG_TILE, DEG_TILE)])
    plsc.subcore_barrier()

    def chunk(m, _):
        r0 = cid * DEG_ROWS_PER_SC + tid * DEG_ROWS_PER_TILE + m * 8
        pltpu.sync_copy(dst2d.at[pl.ds(r0, 8)], stage)
        for j in range(8):
            pltpu.sync_copy(ones_v, deg_sh.at[stage.at[j]], add=True)
        return 0
    lax.fori_loop(0, DEG_CHUNKS, chunk, 0)

    plsc.subcore_barrier()
    pltpu.sync_copy(deg_sh.at[pl.ds(tid * DEG_TILE, DEG_TILE)],
                    deg_out.at[pl.ds(cid * DEG_WORDS + tid * DEG_TILE, DEG_TILE)])


_deg_kernel = pl.kernel(
    _deg_body,
    out_type=jax.ShapeDtypeStruct((NC * DEG_WORDS,), jnp.float32),
    mesh=plsc.VectorSubcoreMesh(
        core_axis_name="c", subcore_axis_name="s", num_cores=NC, num_subcores=NS),
    scratch_types=[
        pltpu.VMEM((DEG_TILE,), jnp.float32),
        pltpu.VMEM((128,), jnp.float32),
        pltpu.VMEM((8, 128), jnp.int32),
        pltpu.VMEM_SHARED((DEG_WORDS,), jnp.float32),
    ],
)


def _scat_body(src2d, dst2d, g, acc_out, zrows, stage_s, stage_d, ldst, rows,
               acc_sh):
    cid = lax.axis_index("c")
    tid = lax.axis_index("s")
    base = cid * LOCAL
    zeros16 = jnp.zeros((LANES,), jnp.float32)

    def zr(r, _):
        zrows[r, pl.ds(0, LANES)] = zeros16
        zrows[r, pl.ds(LANES, LANES)] = zeros16
        return 0
    lax.fori_loop(0, 64, zr, 0)

    def zc(m, _):
        pltpu.sync_copy(zrows, acc_sh.at[pl.ds(tid * ZCHUNKS * 128 + m * 64, 64)])
        return 0
    lax.fori_loop(0, 2 * ZCHUNKS, zc, 0)
    plsc.subcore_barrier()

    def chunk(m, _):
        r0 = tid * ROWS_PER_TILE + m * CROWS
        pltpu.sync_copy(src2d.at[pl.ds(r0, CROWS)], stage_s)
        pltpu.sync_copy(dst2d.at[pl.ds(r0, CROWS)], stage_d)
        for j in range(CROWS):
            for k in range(128 // LANES):
                v = stage_d[j, pl.ds(k * LANES, LANES)]
                ok = (v >= base) & (v < base + LOCAL)
                ldst[j, pl.ds(k * LANES, LANES)] = jnp.where(ok, v - base, DUMMY)
        for j in range(CROWS):
            pltpu.sync_copy(g.at[stage_s.at[j]], rows.at[pl.ds(j * 128, 128)])
        for j in range(CROWS):
            pltpu.sync_copy(rows.at[pl.ds(j * 128, 128)], acc_sh.at[ldst.at[j]],
                            add=True)
        return 0
    lax.fori_loop(0, CHUNKS, chunk, 0)

    plsc.subcore_barrier()
    n_cp = jnp.where(tid < CP_CHUNKS - (CP_CHUNKS // NS) * NS, CP_CHUNKS // NS + 1,
                     CP_CHUNKS // NS)

    def cp(m, _):
        c = m * NS + tid
        pltpu.sync_copy(acc_sh.at[pl.ds(c * CP_ROWS, CP_ROWS)],
                        acc_out.at[pl.ds(cid * LOCAL + c * CP_ROWS, CP_ROWS)])
        return 0
    lax.fori_loop(0, n_cp, cp, 0)


_scat_kernel = pl.kernel(
    _scat_body,
    out_type=jax.ShapeDtypeStruct((N_NODES, DIM), jnp.float32),
    mesh=plsc.VectorSubcoreMesh(
        core_axis_name="c", subcore_axis_name="s", num_cores=NC, num_subcores=NS),
    compiler_params=pltpu.CompilerParams(use_tc_tiling_on_sc=False),
    scratch_types=[
        pltpu.VMEM((64, DIM), jnp.float32),
        pltpu.VMEM((CROWS, 128), jnp.int32),
        pltpu.VMEM((CROWS, 128), jnp.int32),
        pltpu.VMEM((CROWS, 128), jnp.int32),
        pltpu.VMEM((CROWS * 128, DIM), jnp.float32),
        pltpu.VMEM_SHARED((ACC_ROWS, DIM), jnp.float32),
    ],
)


BLK = 2000
GRID = N_NODES // BLK


def _tc1_body(emb_ref, w1_ref, dp_ref, g1_ref, dinv_ref):
    d = dp_ref[...]
    dv = lax.rsqrt(d[0] + d[1] + 1.0)
    h = jnp.dot(emb_ref[...], w1_ref[...], preferred_element_type=jnp.float32)
    g1_ref[...] = h * dv
    dinv_ref[...] = dv


def _tc1(emb, w1, dp):
    return pl.pallas_call(
        _tc1_body,
        grid=(GRID,),
        in_specs=[
            pl.BlockSpec((BLK, DIM), lambda i: (i, 0)),
            pl.BlockSpec((DIM, DIM), lambda i: (0, 0)),
            pl.BlockSpec((NC, BLK, 1), lambda i: (0, i, 0)),
        ],
        out_specs=[
            pl.BlockSpec((BLK, DIM), lambda i: (i, 0)),
            pl.BlockSpec((BLK, 1), lambda i: (i, 0)),
        ],
        out_shape=[
            jax.ShapeDtypeStruct((N_NODES, DIM), jnp.float32),
            jax.ShapeDtypeStruct((N_NODES, 1), jnp.float32),
        ],
    )(emb, w1, dp)


def _tc2_body(acc_ref, g_ref, dinv_ref, b_ref, w_ref, g2_ref):
    dv = dinv_ref[...]
    h = jax.nn.relu(dv * (acc_ref[...] + g_ref[...]) + b_ref[...])
    g2_ref[...] = jnp.dot(h, w_ref[...], preferred_element_type=jnp.float32) * dv


def _tc2(acc, g, dinv, b, w):
    return pl.pallas_call(
        _tc2_body,
        grid=(GRID,),
        in_specs=[
            pl.BlockSpec((BLK, DIM), lambda i: (i, 0)),
            pl.BlockSpec((BLK, DIM), lambda i: (i, 0)),
            pl.BlockSpec((BLK, 1), lambda i: (i, 0)),
            pl.BlockSpec((1, DIM), lambda i: (0, 0)),
            pl.BlockSpec((DIM, DIM), lambda i: (0, 0)),
        ],
        out_specs=pl.BlockSpec((BLK, DIM), lambda i: (i, 0)),
        out_shape=jax.ShapeDtypeStruct((N_NODES, DIM), jnp.float32),
    )(acc, g, dinv, b, w)


def _tc3_body(acc_ref, g_ref, dinv_ref, b_ref, wh_ref, bh_ref, out_ref):
    dv = dinv_ref[...]
    h = jax.nn.relu(dv * (acc_ref[...] + g_ref[...]) + b_ref[...])
    z = jnp.dot(h, wh_ref[...], preferred_element_type=jnp.float32) + bh_ref[...]
    out_ref[...] = jax.nn.sigmoid(z)


def _tc3(acc, g, dinv, b, wh, bh):
    return pl.pallas_call(
        _tc3_body,
        grid=(GRID,),
        in_specs=[
            pl.BlockSpec((BLK, DIM), lambda i: (i, 0)),
            pl.BlockSpec((BLK, DIM), lambda i: (i, 0)),
            pl.BlockSpec((BLK, 1), lambda i: (i, 0)),
            pl.BlockSpec((1, DIM), lambda i: (0, 0)),
            pl.BlockSpec((DIM, 1), lambda i: (0, 0)),
            pl.BlockSpec((1, 1), lambda i: (0, 0)),
        ],
        out_specs=pl.BlockSpec((BLK, 1), lambda i: (i, 0)),
        out_shape=jax.ShapeDtypeStruct((N_NODES, 1), jnp.float32),
    )(acc, g, dinv, b, wh, bh)


@jax.jit
def kernel(edge_index, emb, W1, b1, W2, b2, Wh, bh):
    src = edge_index[0].astype(jnp.int32)
    dst = edge_index[1].astype(jnp.int32)
    pad = EP - N_EDGES
    src2d = jnp.concatenate(
        [src, jnp.zeros((pad,), jnp.int32)]).reshape(ROWS, 128)
    dst2d = jnp.concatenate(
        [dst, jnp.full((pad,), N_NODES, jnp.int32)]).reshape(ROWS, 128)

    deg2 = _deg_kernel(dst2d).reshape(NC, DEG_WORDS)
    dp = deg2[:, :N_NODES].reshape(NC, N_NODES, 1)

    g1, dinv = _tc1(emb, W1, dp)
    acc1 = _scat_kernel(src2d, dst2d, g1)
    g2 = _tc2(acc1, g1, dinv, b1.reshape(1, DIM), W2)
    acc2 = _scat_kernel(src2d, dst2d, g2)
    out = _tc3(acc2, g2, dinv, b2.reshape(1, DIM), Wh, bh.reshape(1, 1))
    return out.reshape(N_NODES)


# speedup vs baseline: 12.4936x; 1.1208x over previous
import functools

import jax
import jax.numpy as jnp
from jax import lax
from jax.experimental import pallas as pl
from jax.experimental.pallas import tpu as pltpu
from jax.experimental.pallas import tpu_sc as plsc

N_NODES = 100_000
DIM = 32
N_EDGES = 1_600_000

NC = 2
NS = 16
LANES = 16

ROWS = 12_800
EP = ROWS * 128

DEG_WORDS = 100_352
DEG_TILE = DEG_WORDS // NS
DEG_ROWS_PER_SC = ROWS // NC
DEG_ROWS_PER_TILE = DEG_ROWS_PER_SC // NS
DEG_CHUNKS = DEG_ROWS_PER_TILE // 8

LOCAL = N_NODES // NC
ACC_ROWS = 50_176
DUMMY = LOCAL
ZCOPIES = ACC_ROWS // NS // 64
ROWS_PER_TILE = ROWS // NS
CR = 2
CHUNKS = ROWS_PER_TILE // CR
NPAIR = CHUNKS // 2
CP_ROWS = 400
CP_CHUNKS = LOCAL // CP_ROWS


def _deg_body(dst2d, deg_out, zbuf, ones_v, stage, deg_sh):
    cid = lax.axis_index("c")
    tid = lax.axis_index("s")
    zeros16 = jnp.zeros((LANES,), jnp.float32)
    ones16 = jnp.ones((LANES,), jnp.float32)

    def zb(i, _):
        zbuf[pl.ds(i * LANES, LANES)] = zeros16
        return 0
    lax.fori_loop(0, DEG_TILE // LANES, zb, 0)
    for i in range(128 // LANES):
        ones_v[pl.ds(i * LANES, LANES)] = ones16

    pltpu.sync_copy(zbuf, deg_sh.at[pl.ds(tid * DEG_TILE, DEG_TILE)])
    plsc.subcore_barrier()

    def chunk(m, _):
        r0 = cid * DEG_ROWS_PER_SC + tid * DEG_ROWS_PER_TILE + m * 8
        pltpu.sync_copy(dst2d.at[pl.ds(r0, 8)], stage)
        for j in range(8):
            pltpu.sync_copy(ones_v, deg_sh.at[stage.at[j]], add=True)
        return 0
    lax.fori_loop(0, DEG_CHUNKS, chunk, 0)

    plsc.subcore_barrier()
    pltpu.sync_copy(deg_sh.at[pl.ds(tid * DEG_TILE, DEG_TILE)],
                    deg_out.at[pl.ds(cid * DEG_WORDS + tid * DEG_TILE, DEG_TILE)])


_deg_kernel = pl.kernel(
    _deg_body,
    out_type=jax.ShapeDtypeStruct((NC * DEG_WORDS,), jnp.float32),
    mesh=plsc.VectorSubcoreMesh(
        core_axis_name="c", subcore_axis_name="s", num_cores=NC, num_subcores=NS),
    scratch_types=[
        pltpu.VMEM((DEG_TILE,), jnp.float32),
        pltpu.VMEM((128,), jnp.float32),
        pltpu.VMEM((8, 128), jnp.int32),
        pltpu.VMEM_SHARED((DEG_WORDS,), jnp.float32),
    ],
)


def _scat_body(src2d, dst2d, g, acc_out, zrows, stage_s, stage_d, ldst, rows,
               acc_sh, sem_i, sem_g, sem_s0, sem_s1):
    cid = lax.axis_index("c")
    tid = lax.axis_index("s")
    base = cid * LOCAL
    zeros16 = jnp.zeros((LANES,), jnp.float32)

    def zr(r, _):
        zrows[r, pl.ds(0, LANES)] = zeros16
        zrows[r, pl.ds(LANES, LANES)] = zeros16
        return 0
    lax.fori_loop(0, 64, zr, 0)

    def zc(m, _):
        pltpu.sync_copy(zrows, acc_sh.at[pl.ds(tid * ZCOPIES * 64 + m * 64, 64)])
        return 0
    lax.fori_loop(0, ZCOPIES, zc, 0)
    plsc.subcore_barrier()

    t0 = tid * ROWS_PER_TILE

    def stage_idx(r0, p):
        pltpu.async_copy(src2d.at[pl.ds(r0, CR)], stage_s.at[p], sem_i)
        pltpu.async_copy(dst2d.at[pl.ds(r0, CR)], stage_d.at[p], sem_i)

    def wait_idx(r0, p):
        pltpu.make_async_copy(src2d.at[pl.ds(r0, CR)], stage_s.at[p], sem_i).wait()
        pltpu.make_async_copy(dst2d.at[pl.ds(r0, CR)], stage_d.at[p], sem_i).wait()

    def drain_scat(p, sem):
        for j in range(CR):
            pltpu.make_async_copy(rows.at[p, pl.ds(j * 128, 128)],
                                  acc_sh.at[ldst.at[p, j]], sem).wait()

    def do_chunk(r0, p, sem):
        for j in range(CR):
            pltpu.async_copy(g.at[stage_s.at[p, j]],
                             rows.at[p, pl.ds(j * 128, 128)], sem_g)
        for j in range(CR):
            for k in range(128 // LANES):
                v = stage_d[p, j, pl.ds(k * LANES, LANES)]
                ok = (v >= base) & (v < base + LOCAL)
                ldst[p, j, pl.ds(k * LANES, LANES)] = jnp.where(ok, v - base, DUMMY)
        for j in range(CR):
            pltpu.make_async_copy(g.at[stage_s.at[p, j]],
                                  rows.at[p, pl.ds(j * 128, 128)], sem_g).wait()
        for j in range(CR):
            pltpu.async_copy(rows.at[p, pl.ds(j * 128, 128)],
                             acc_sh.at[ldst.at[p, j]], sem, add=True)

    stage_idx(t0, 0)

    def pair(n, _):
        r0 = t0 + n * 2 * CR
        r1 = r0 + CR
        wait_idx(r0, 0)
        stage_idx(r1, 1)

        @pl.when(n >= 1)
        def _():
            drain_scat(0, sem_s0)
        do_chunk(r0, 0, sem_s0)

        wait_idx(r1, 1)

        @pl.when(n < NPAIR - 1)
        def _():
            stage_idx(r1 + CR, 0)

        @pl.when(n >= 1)
        def _():
            drain_scat(1, sem_s1)
        do_chunk(r1, 1, sem_s1)
        return 0
    lax.fori_loop(0, NPAIR, pair, 0)

    drain_scat(0, sem_s0)
    drain_scat(1, sem_s1)
    plsc.subcore_barrier()
    n_cp = jnp.where(tid < CP_CHUNKS - (CP_CHUNKS // NS) * NS, CP_CHUNKS // NS + 1,
                     CP_CHUNKS // NS)

    def cp(m, _):
        c = m * NS + tid
        pltpu.sync_copy(acc_sh.at[pl.ds(c * CP_ROWS, CP_ROWS)],
                        acc_out.at[pl.ds(cid * LOCAL + c * CP_ROWS, CP_ROWS)])
        return 0
    lax.fori_loop(0, n_cp, cp, 0)


_scat_kernel = pl.kernel(
    _scat_body,
    out_type=jax.ShapeDtypeStruct((N_NODES, DIM), jnp.float32),
    mesh=plsc.VectorSubcoreMesh(
        core_axis_name="c", subcore_axis_name="s", num_cores=NC, num_subcores=NS),
    compiler_params=pltpu.CompilerParams(use_tc_tiling_on_sc=False),
    scratch_types=[
        pltpu.VMEM((64, DIM), jnp.float32),
        pltpu.VMEM((2, CR, 128), jnp.int32),
        pltpu.VMEM((2, CR, 128), jnp.int32),
        pltpu.VMEM((2, CR, 128), jnp.int32),
        pltpu.VMEM((2, CR * 128, DIM), jnp.float32),
        pltpu.VMEM_SHARED((ACC_ROWS, DIM), jnp.float32),
        pltpu.SemaphoreType.DMA,
        pltpu.SemaphoreType.DMA,
        pltpu.SemaphoreType.DMA,
        pltpu.SemaphoreType.DMA,
    ],
)


BLK = 2000
GRID = N_NODES // BLK


def _tc1_body(emb_ref, w1_ref, dp_ref, g1_ref, dinv_ref):
    d = dp_ref[...]
    dv = lax.rsqrt(d[0] + d[1] + 1.0)
    h = jnp.dot(emb_ref[...], w1_ref[...], preferred_element_type=jnp.float32)
    g1_ref[...] = h * dv
    dinv_ref[...] = dv


def _tc1(emb, w1, dp):
    return pl.pallas_call(
        _tc1_body,
        grid=(GRID,),
        in_specs=[
            pl.BlockSpec((BLK, DIM), lambda i: (i, 0)),
            pl.BlockSpec((DIM, DIM), lambda i: (0, 0)),
            pl.BlockSpec((NC, BLK, 1), lambda i: (0, i, 0)),
        ],
        out_specs=[
            pl.BlockSpec((BLK, DIM), lambda i: (i, 0)),
            pl.BlockSpec((BLK, 1), lambda i: (i, 0)),
        ],
        out_shape=[
            jax.ShapeDtypeStruct((N_NODES, DIM), jnp.float32),
            jax.ShapeDtypeStruct((N_NODES, 1), jnp.float32),
        ],
    )(emb, w1, dp)


def _tc2_body(acc_ref, g_ref, dinv_ref, b_ref, w_ref, g2_ref):
    dv = dinv_ref[...]
    h = jax.nn.relu(dv * (acc_ref[...] + g_ref[...]) + b_ref[...])
    g2_ref[...] = jnp.dot(h, w_ref[...], preferred_element_type=jnp.float32) * dv


def _tc2(acc, g, dinv, b, w):
    return pl.pallas_call(
        _tc2_body,
        grid=(GRID,),
        in_specs=[
            pl.BlockSpec((BLK, DIM), lambda i: (i, 0)),
            pl.BlockSpec((BLK, DIM), lambda i: (i, 0)),
            pl.BlockSpec((BLK, 1), lambda i: (i, 0)),
            pl.BlockSpec((1, DIM), lambda i: (0, 0)),
            pl.BlockSpec((DIM, DIM), lambda i: (0, 0)),
        ],
        out_specs=pl.BlockSpec((BLK, DIM), lambda i: (i, 0)),
        out_shape=jax.ShapeDtypeStruct((N_NODES, DIM), jnp.float32),
    )(acc, g, dinv, b, w)


def _tc3_body(acc_ref, g_ref, dinv_ref, b_ref, wh_ref, bh_ref, out_ref):
    dv = dinv_ref[...]
    h = jax.nn.relu(dv * (acc_ref[...] + g_ref[...]) + b_ref[...])
    z = jnp.dot(h, wh_ref[...], preferred_element_type=jnp.float32) + bh_ref[...]
    out_ref[...] = jax.nn.sigmoid(z)


def _tc3(acc, g, dinv, b, wh, bh):
    return pl.pallas_call(
        _tc3_body,
        grid=(GRID,),
        in_specs=[
            pl.BlockSpec((BLK, DIM), lambda i: (i, 0)),
            pl.BlockSpec((BLK, DIM), lambda i: (i, 0)),
            pl.BlockSpec((BLK, 1), lambda i: (i, 0)),
            pl.BlockSpec((1, DIM), lambda i: (0, 0)),
            pl.BlockSpec((DIM, 1), lambda i: (0, 0)),
            pl.BlockSpec((1, 1), lambda i: (0, 0)),
        ],
        out_specs=pl.BlockSpec((BLK, 1), lambda i: (i, 0)),
        out_shape=jax.ShapeDtypeStruct((N_NODES, 1), jnp.float32),
    )(acc, g, dinv, b, wh, bh)


@jax.jit
def kernel(edge_index, emb, W1, b1, W2, b2, Wh, bh):
    src = edge_index[0].astype(jnp.int32)
    dst = edge_index[1].astype(jnp.int32)
    pad = EP - N_EDGES
    src2d = jnp.concatenate(
        [src, jnp.zeros((pad,), jnp.int32)]).reshape(ROWS, 128)
    dst2d = jnp.concatenate(
        [dst, jnp.full((pad,), N_NODES, jnp.int32)]).reshape(ROWS, 128)

    deg2 = _deg_kernel(dst2d).reshape(NC, DEG_WORDS)
    dp = deg2[:, :N_NODES].reshape(NC, N_NODES, 1)

    g1, dinv = _tc1(emb, W1, dp)
    acc1 = _scat_kernel(src2d, dst2d, g1)
    g2 = _tc2(acc1, g1, dinv, b1.reshape(1, DIM), W2)
    acc2 = _scat_kernel(src2d, dst2d, g2)
    out = _tc3(acc2, g2, dinv, b2.reshape(1, DIM), Wh, bh.reshape(1, 1))
    return out.reshape(N_NODES)


# speedup vs baseline: 19.2499x; 1.5408x over previous
import functools

import jax
import jax.numpy as jnp
from jax import lax
from jax.experimental import pallas as pl
from jax.experimental.pallas import tpu as pltpu
from jax.experimental.pallas import tpu_sc as plsc

N_NODES = 100_000
DIM = 32
N_EDGES = 1_600_000

NC = 2
NS = 16
LANES = 16

ROWS = 12_800
EP = ROWS * 128

DEG_WORDS = 100_352
DEG_TILE = DEG_WORDS // NS
DEG_ROWS_PER_SC = ROWS // NC
DEG_ROWS_PER_TILE = DEG_ROWS_PER_SC // NS
DEG_CHUNKS = DEG_ROWS_PER_TILE // 8

LOCAL = N_NODES // NC
ACC_ROWS = 50_176
DUMMY = LOCAL
ZCOPIES = ACC_ROWS // NS // 64
ROWS_PER_TILE = ROWS // NS
CR = 2
CHUNKS = ROWS_PER_TILE // CR
NPAIR = CHUNKS // 2
CP_ROWS = 400
CP_CHUNKS = LOCAL // CP_ROWS


def _deg_body(dst2d, deg_out, zbuf, ones_v, stage, deg_sh):
    cid = lax.axis_index("c")
    tid = lax.axis_index("s")
    zeros16 = jnp.zeros((LANES,), jnp.float32)
    ones16 = jnp.ones((LANES,), jnp.float32)

    def zb(i, _):
        zbuf[pl.ds(i * LANES, LANES)] = zeros16
        return 0
    lax.fori_loop(0, DEG_TILE // LANES, zb, 0)
    for i in range(128 // LANES):
        ones_v[pl.ds(i * LANES, LANES)] = ones16

    pltpu.sync_copy(zbuf, deg_sh.at[pl.ds(tid * DEG_TILE, DEG_TILE)])
    plsc.subcore_barrier()

    def chunk(m, _):
        r0 = cid * DEG_ROWS_PER_SC + tid * DEG_ROWS_PER_TILE + m * 8
        pltpu.sync_copy(dst2d.at[pl.ds(r0, 8)], stage)
        for j in range(8):
            pltpu.sync_copy(ones_v, deg_sh.at[stage.at[j]], add=True)
        return 0
    lax.fori_loop(0, DEG_CHUNKS, chunk, 0)

    plsc.subcore_barrier()
    pltpu.sync_copy(deg_sh.at[pl.ds(tid * DEG_TILE, DEG_TILE)],
                    deg_out.at[pl.ds(cid * DEG_WORDS + tid * DEG_TILE, DEG_TILE)])


_deg_kernel = pl.kernel(
    _deg_body,
    out_type=jax.ShapeDtypeStruct((NC * DEG_WORDS,), jnp.float32),
    mesh=plsc.VectorSubcoreMesh(
        core_axis_name="c", subcore_axis_name="s", num_cores=NC, num_subcores=NS),
    scratch_types=[
        pltpu.VMEM((DEG_TILE,), jnp.float32),
        pltpu.VMEM((128,), jnp.float32),
        pltpu.VMEM((8, 128), jnp.int32),
        pltpu.VMEM_SHARED((DEG_WORDS,), jnp.float32),
    ],
)


def _scat_body(src2d, dst2d, g, acc_out, zrows, stage_s, stage_d, ldst, rows,
               acc_sh, sem_i, sem_g, sem_s0, sem_s1):
    cid = lax.axis_index("c")
    tid = lax.axis_index("s")
    base = cid * LOCAL
    zeros32 = jnp.zeros((2 * LANES,), jnp.bfloat16)

    def zr(r, _):
        zrows[r, pl.ds(0, 2 * LANES)] = zeros32
        return 0
    lax.fori_loop(0, 64, zr, 0)

    def zc(m, _):
        pltpu.sync_copy(zrows, acc_sh.at[pl.ds(tid * ZCOPIES * 64 + m * 64, 64)])
        return 0
    lax.fori_loop(0, ZCOPIES, zc, 0)
    plsc.subcore_barrier()

    t0 = tid * ROWS_PER_TILE

    def stage_idx(r0, p):
        pltpu.async_copy(src2d.at[pl.ds(r0, CR)], stage_s.at[p], sem_i)
        pltpu.async_copy(dst2d.at[pl.ds(r0, CR)], stage_d.at[p], sem_i)

    def wait_idx(r0, p):
        pltpu.make_async_copy(src2d.at[pl.ds(r0, CR)], stage_s.at[p], sem_i).wait()
        pltpu.make_async_copy(dst2d.at[pl.ds(r0, CR)], stage_d.at[p], sem_i).wait()

    def drain_scat(p, sem):
        for j in range(CR):
            pltpu.make_async_copy(rows.at[p, pl.ds(j * 128, 128)],
                                  acc_sh.at[ldst.at[p, j]], sem).wait()

    def do_chunk(r0, p, sem):
        for j in range(CR):
            pltpu.async_copy(g.at[stage_s.at[p, j]],
                             rows.at[p, pl.ds(j * 128, 128)], sem_g)
        for j in range(CR):
            for k in range(128 // LANES):
                v = stage_d[p, j, pl.ds(k * LANES, LANES)]
                ok = (v >= base) & (v < base + LOCAL)
                ldst[p, j, pl.ds(k * LANES, LANES)] = jnp.where(ok, v - base, DUMMY)
        for j in range(CR):
            pltpu.make_async_copy(g.at[stage_s.at[p, j]],
                                  rows.at[p, pl.ds(j * 128, 128)], sem_g).wait()
        for j in range(CR):
            pltpu.async_copy(rows.at[p, pl.ds(j * 128, 128)],
                             acc_sh.at[ldst.at[p, j]], sem, add=True)

    stage_idx(t0, 0)

    def pair(n, _):
        r0 = t0 + n * 2 * CR
        r1 = r0 + CR
        wait_idx(r0, 0)
        stage_idx(r1, 1)

        @pl.when(n >= 1)
        def _():
            drain_scat(0, sem_s0)
        do_chunk(r0, 0, sem_s0)

        wait_idx(r1, 1)

        @pl.when(n < NPAIR - 1)
        def _():
            stage_idx(r1 + CR, 0)

        @pl.when(n >= 1)
        def _():
            drain_scat(1, sem_s1)
        do_chunk(r1, 1, sem_s1)
        return 0
    lax.fori_loop(0, NPAIR, pair, 0)

    drain_scat(0, sem_s0)
    drain_scat(1, sem_s1)
    plsc.subcore_barrier()
    n_cp = jnp.where(tid < CP_CHUNKS - (CP_CHUNKS // NS) * NS, CP_CHUNKS // NS + 1,
                     CP_CHUNKS // NS)

    def cp(m, _):
        c = m * NS + tid
        pltpu.sync_copy(acc_sh.at[pl.ds(c * CP_ROWS, CP_ROWS)],
                        acc_out.at[pl.ds(cid * LOCAL + c * CP_ROWS, CP_ROWS)])
        return 0
    lax.fori_loop(0, n_cp, cp, 0)


_scat_kernel = pl.kernel(
    _scat_body,
    out_type=jax.ShapeDtypeStruct((N_NODES, DIM), jnp.bfloat16),
    mesh=plsc.VectorSubcoreMesh(
        core_axis_name="c", subcore_axis_name="s", num_cores=NC, num_subcores=NS),
    compiler_params=pltpu.CompilerParams(use_tc_tiling_on_sc=False),
    scratch_types=[
        pltpu.VMEM((64, DIM), jnp.bfloat16),
        pltpu.VMEM((2, CR, 128), jnp.int32),
        pltpu.VMEM((2, CR, 128), jnp.int32),
        pltpu.VMEM((2, CR, 128), jnp.int32),
        pltpu.VMEM((2, CR * 128, DIM), jnp.bfloat16),
        pltpu.VMEM_SHARED((ACC_ROWS, DIM), jnp.bfloat16),
        pltpu.SemaphoreType.DMA,
        pltpu.SemaphoreType.DMA,
        pltpu.SemaphoreType.DMA,
        pltpu.SemaphoreType.DMA,
    ],
)


BLK = 2000
GRID = N_NODES // BLK


def _tc1_body(emb_ref, w1_ref, dp_ref, g1_ref, dinv_ref):
    d = dp_ref[...]
    dv = lax.rsqrt(d[0] + d[1] + 1.0)
    h = jnp.dot(emb_ref[...], w1_ref[...], preferred_element_type=jnp.float32)
    g1_ref[...] = (h * dv).astype(jnp.bfloat16)
    dinv_ref[...] = dv


def _tc1(emb, w1, dp):
    return pl.pallas_call(
        _tc1_body,
        grid=(GRID,),
        in_specs=[
            pl.BlockSpec((BLK, DIM), lambda i: (i, 0)),
            pl.BlockSpec((DIM, DIM), lambda i: (0, 0)),
            pl.BlockSpec((NC, BLK, 1), lambda i: (0, i, 0)),
        ],
        out_specs=[
            pl.BlockSpec((BLK, DIM), lambda i: (i, 0)),
            pl.BlockSpec((BLK, 1), lambda i: (i, 0)),
        ],
        out_shape=[
            jax.ShapeDtypeStruct((N_NODES, DIM), jnp.bfloat16),
            jax.ShapeDtypeStruct((N_NODES, 1), jnp.float32),
        ],
    )(emb, w1, dp)


def _tc2_body(acc_ref, g_ref, dinv_ref, b_ref, w_ref, g2_ref):
    dv = dinv_ref[...]
    s = acc_ref[...].astype(jnp.float32) + g_ref[...].astype(jnp.float32)
    h = jax.nn.relu(dv * s + b_ref[...])
    g2 = jnp.dot(h, w_ref[...], preferred_element_type=jnp.float32) * dv
    g2_ref[...] = g2.astype(jnp.bfloat16)


def _tc2(acc, g, dinv, b, w):
    return pl.pallas_call(
        _tc2_body,
        grid=(GRID,),
        in_specs=[
            pl.BlockSpec((BLK, DIM), lambda i: (i, 0)),
            pl.BlockSpec((BLK, DIM), lambda i: (i, 0)),
            pl.BlockSpec((BLK, 1), lambda i: (i, 0)),
            pl.BlockSpec((1, DIM), lambda i: (0, 0)),
            pl.BlockSpec((DIM, DIM), lambda i: (0, 0)),
        ],
        out_specs=pl.BlockSpec((BLK, DIM), lambda i: (i, 0)),
        out_shape=jax.ShapeDtypeStruct((N_NODES, DIM), jnp.bfloat16),
    )(acc, g, dinv, b, w)


def _tc3_body(acc_ref, g_ref, dinv_ref, b_ref, wh_ref, bh_ref, out_ref):
    dv = dinv_ref[...]
    s = acc_ref[...].astype(jnp.float32) + g_ref[...].astype(jnp.float32)
    h = jax.nn.relu(dv * s + b_ref[...])
    z = jnp.dot(h, wh_ref[...], preferred_element_type=jnp.float32) + bh_ref[...]
    out_ref[...] = jax.nn.sigmoid(z)


def _tc3(acc, g, dinv, b, wh, bh):
    return pl.pallas_call(
        _tc3_body,
        grid=(GRID,),
        in_specs=[
            pl.BlockSpec((BLK, DIM), lambda i: (i, 0)),
            pl.BlockSpec((BLK, DIM), lambda i: (i, 0)),
            pl.BlockSpec((BLK, 1), lambda i: (i, 0)),
            pl.BlockSpec((1, DIM), lambda i: (0, 0)),
            pl.BlockSpec((DIM, 1), lambda i: (0, 0)),
            pl.BlockSpec((1, 1), lambda i: (0, 0)),
        ],
        out_specs=pl.BlockSpec((BLK, 1), lambda i: (i, 0)),
        out_shape=jax.ShapeDtypeStruct((N_NODES, 1), jnp.float32),
    )(acc, g, dinv, b, wh, bh)


@jax.jit
def kernel(edge_index, emb, W1, b1, W2, b2, Wh, bh):
    src = edge_index[0].astype(jnp.int32)
    dst = edge_index[1].astype(jnp.int32)
    pad = EP - N_EDGES
    src2d = jnp.concatenate(
        [src, jnp.zeros((pad,), jnp.int32)]).reshape(ROWS, 128)
    dst2d = jnp.concatenate(
        [dst, jnp.full((pad,), N_NODES, jnp.int32)]).reshape(ROWS, 128)

    deg2 = _deg_kernel(dst2d).reshape(NC, DEG_WORDS)
    dp = deg2[:, :N_NODES].reshape(NC, N_NODES, 1)

    g1, dinv = _tc1(emb, W1, dp)
    acc1 = _scat_kernel(src2d, dst2d, g1)
    g2 = _tc2(acc1, g1, dinv, b1.reshape(1, DIM), W2)
    acc2 = _scat_kernel(src2d, dst2d, g2)
    out = _tc3(acc2, g2, dinv, b2.reshape(1, DIM), Wh, bh.reshape(1, 1))
    return out.reshape(N_NODES)


# speedup vs baseline: 25.0712x; 1.3024x over previous
import functools

import jax
import jax.numpy as jnp
from jax import lax
from jax.experimental import pallas as pl
from jax.experimental.pallas import tpu as pltpu
from jax.experimental.pallas import tpu_sc as plsc

N_NODES = 100_000
DIM = 32
N_EDGES = 1_600_000

NC = 2
NS = 16
LANES = 16

ROWS = 12_800
EP = ROWS * 128

DEG_WORDS = 100_352
DEG_TILE = DEG_WORDS // NS
DEG_ROWS_PER_SC = ROWS // NC
DEG_ROWS_PER_TILE = DEG_ROWS_PER_SC // NS
DEG_CHUNKS = DEG_ROWS_PER_TILE // 8

ACC_ROWS = DEG_WORDS
ACC_TILE = ACC_ROWS // NS
ZCOPIES = ACC_TILE // 64
ROWS_PER_TILE = ROWS // NC // NS
CR = 2
CHUNKS = ROWS_PER_TILE // CR
NPAIR = CHUNKS // 2


def _deg_body(dst2d, deg_out, zbuf, ones_v, stage, deg_sh):
    cid = lax.axis_index("c")
    tid = lax.axis_index("s")
    zeros16 = jnp.zeros((LANES,), jnp.float32)
    ones16 = jnp.ones((LANES,), jnp.float32)

    def zb(i, _):
        zbuf[pl.ds(i * LANES, LANES)] = zeros16
        return 0
    lax.fori_loop(0, DEG_TILE // LANES, zb, 0)
    for i in range(128 // LANES):
        ones_v[pl.ds(i * LANES, LANES)] = ones16

    pltpu.sync_copy(zbuf, deg_sh.at[pl.ds(tid * DEG_TILE, DEG_TILE)])
    plsc.subcore_barrier()

    def chunk(m, _):
        r0 = cid * DEG_ROWS_PER_SC + tid * DEG_ROWS_PER_TILE + m * 8
        pltpu.sync_copy(dst2d.at[pl.ds(r0, 8)], stage)
        for j in range(8):
            pltpu.sync_copy(ones_v, deg_sh.at[stage.at[j]], add=True)
        return 0
    lax.fori_loop(0, DEG_CHUNKS, chunk, 0)

    plsc.subcore_barrier()
    pltpu.sync_copy(deg_sh.at[pl.ds(tid * DEG_TILE, DEG_TILE)],
                    deg_out.at[pl.ds(cid * DEG_WORDS + tid * DEG_TILE, DEG_TILE)])


_deg_kernel = pl.kernel(
    _deg_body,
    out_type=jax.ShapeDtypeStruct((NC * DEG_WORDS,), jnp.float32),
    mesh=plsc.VectorSubcoreMesh(
        core_axis_name="c", subcore_axis_name="s", num_cores=NC, num_subcores=NS),
    scratch_types=[
        pltpu.VMEM((DEG_TILE,), jnp.float32),
        pltpu.VMEM((128,), jnp.float32),
        pltpu.VMEM((8, 128), jnp.int32),
        pltpu.VMEM_SHARED((DEG_WORDS,), jnp.float32),
    ],
)


def _scat_body(src2d, dst2d, g, acc_out, zrows, stage_s, stage_d, ldst, rows,
               acc_sh, sem_i, sem_g, sem_s0, sem_s1):
    cid = lax.axis_index("c")
    tid = lax.axis_index("s")
    zeros32 = jnp.zeros((2 * LANES,), jnp.bfloat16)

    def zr(r, _):
        zrows[r, pl.ds(0, 2 * LANES)] = zeros32
        return 0
    lax.fori_loop(0, 64, zr, 0)

    def zc(m, _):
        pltpu.sync_copy(zrows, acc_sh.at[pl.ds(tid * ACC_TILE + m * 64, 64)])
        return 0
    lax.fori_loop(0, ZCOPIES, zc, 0)
    plsc.subcore_barrier()

    t0 = (cid * NS + tid) * ROWS_PER_TILE

    def stage_idx(r0, p):
        pltpu.async_copy(src2d.at[pl.ds(r0, CR)], stage_s.at[p], sem_i)
        pltpu.async_copy(dst2d.at[pl.ds(r0, CR)], stage_d.at[p], sem_i)

    def wait_idx(r0, p):
        pltpu.make_async_copy(src2d.at[pl.ds(r0, CR)], stage_s.at[p], sem_i).wait()
        pltpu.make_async_copy(dst2d.at[pl.ds(r0, CR)], stage_d.at[p], sem_i).wait()

    def drain_scat(p, sem):
        for j in range(CR):
            pltpu.make_async_copy(rows.at[p, pl.ds(j * 128, 128)],
                                  acc_sh.at[ldst.at[p, j]], sem).wait()

    def do_chunk(r0, p, sem):
        for j in range(CR):
            pltpu.async_copy(g.at[stage_s.at[p, j]],
                             rows.at[p, pl.ds(j * 128, 128)], sem_g)
        for j in range(CR):
            for k in range(128 // LANES):
                ldst[p, j, pl.ds(k * LANES, LANES)] = (
                    stage_d[p, j, pl.ds(k * LANES, LANES)])
        for j in range(CR):
            pltpu.make_async_copy(g.at[stage_s.at[p, j]],
                                  rows.at[p, pl.ds(j * 128, 128)], sem_g).wait()
        for j in range(CR):
            pltpu.async_copy(rows.at[p, pl.ds(j * 128, 128)],
                             acc_sh.at[ldst.at[p, j]], sem, add=True)

    stage_idx(t0, 0)

    def pair(n, _):
        r0 = t0 + n * 2 * CR
        r1 = r0 + CR
        wait_idx(r0, 0)
        stage_idx(r1, 1)

        @pl.when(n >= 1)
        def _():
            drain_scat(0, sem_s0)
        do_chunk(r0, 0, sem_s0)

        wait_idx(r1, 1)

        @pl.when(n < NPAIR - 1)
        def _():
            stage_idx(r1 + CR, 0)

        @pl.when(n >= 1)
        def _():
            drain_scat(1, sem_s1)
        do_chunk(r1, 1, sem_s1)
        return 0
    lax.fori_loop(0, NPAIR, pair, 0)

    drain_scat(0, sem_s0)
    drain_scat(1, sem_s1)
    plsc.subcore_barrier()
    pltpu.sync_copy(acc_sh.at[pl.ds(tid * ACC_TILE, ACC_TILE)],
                    acc_out.at[pl.ds(cid * ACC_ROWS + tid * ACC_TILE, ACC_TILE)])


_scat_kernel = pl.kernel(
    _scat_body,
    out_type=jax.ShapeDtypeStruct((NC * ACC_ROWS, DIM), jnp.bfloat16),
    mesh=plsc.VectorSubcoreMesh(
        core_axis_name="c", subcore_axis_name="s", num_cores=NC, num_subcores=NS),
    compiler_params=pltpu.CompilerParams(use_tc_tiling_on_sc=False),
    scratch_types=[
        pltpu.VMEM((64, DIM), jnp.bfloat16),
        pltpu.VMEM((2, CR, 128), jnp.int32),
        pltpu.VMEM((2, CR, 128), jnp.int32),
        pltpu.VMEM((2, CR, 128), jnp.int32),
        pltpu.VMEM((2, CR * 128, DIM), jnp.bfloat16),
        pltpu.VMEM_SHARED((ACC_ROWS, DIM), jnp.bfloat16),
        pltpu.SemaphoreType.DMA,
        pltpu.SemaphoreType.DMA,
        pltpu.SemaphoreType.DMA,
        pltpu.SemaphoreType.DMA,
    ],
)


BLK = 2000
GRID = N_NODES // BLK


def _tc1_body(emb_ref, w1_ref, dp_ref, g1_ref, dinv_ref):
    d = dp_ref[...]
    dv = lax.rsqrt(d[0] + d[1] + 1.0)
    h = jnp.dot(emb_ref[...], w1_ref[...], preferred_element_type=jnp.float32)
    g1_ref[...] = (h * dv).astype(jnp.bfloat16)
    dinv_ref[...] = dv


def _tc1(emb, w1, dp):
    return pl.pallas_call(
        _tc1_body,
        grid=(GRID,),
        in_specs=[
            pl.BlockSpec((BLK, DIM), lambda i: (i, 0)),
            pl.BlockSpec((DIM, DIM), lambda i: (0, 0)),
            pl.BlockSpec((NC, BLK, 1), lambda i: (0, i, 0)),
        ],
        out_specs=[
            pl.BlockSpec((BLK, DIM), lambda i: (i, 0)),
            pl.BlockSpec((BLK, 1), lambda i: (i, 0)),
        ],
        out_shape=[
            jax.ShapeDtypeStruct((N_NODES, DIM), jnp.bfloat16),
            jax.ShapeDtypeStruct((N_NODES, 1), jnp.float32),
        ],
    )(emb, w1, dp)


def _tc2_body(acc_ref, g_ref, dinv_ref, b_ref, w_ref, g2_ref):
    dv = dinv_ref[...]
    a = acc_ref[...]
    s = (a[0].astype(jnp.float32) + a[1].astype(jnp.float32)
         + g_ref[...].astype(jnp.float32))
    h = jax.nn.relu(dv * s + b_ref[...])
    g2 = jnp.dot(h, w_ref[...], preferred_element_type=jnp.float32) * dv
    g2_ref[...] = g2.astype(jnp.bfloat16)


def _tc2(acc, g, dinv, b, w):
    return pl.pallas_call(
        _tc2_body,
        grid=(GRID,),
        in_specs=[
            pl.BlockSpec((NC, BLK, DIM), lambda i: (0, i, 0)),
            pl.BlockSpec((BLK, DIM), lambda i: (i, 0)),
            pl.BlockSpec((BLK, 1), lambda i: (i, 0)),
            pl.BlockSpec((1, DIM), lambda i: (0, 0)),
            pl.BlockSpec((DIM, DIM), lambda i: (0, 0)),
        ],
        out_specs=pl.BlockSpec((BLK, DIM), lambda i: (i, 0)),
        out_shape=jax.ShapeDtypeStruct((N_NODES, DIM), jnp.bfloat16),
    )(acc, g, dinv, b, w)


def _tc3_body(acc_ref, g_ref, dinv_ref, b_ref, wh_ref, bh_ref, out_ref):
    dv = dinv_ref[...]
    a = acc_ref[...]
    s = (a[0].astype(jnp.float32) + a[1].astype(jnp.float32)
         + g_ref[...].astype(jnp.float32))
    h = jax.nn.relu(dv * s + b_ref[...])
    z = jnp.dot(h, wh_ref[...], preferred_element_type=jnp.float32) + bh_ref[...]
    out_ref[...] = jax.nn.sigmoid(z)


def _tc3(acc, g, dinv, b, wh, bh):
    return pl.pallas_call(
        _tc3_body,
        grid=(GRID,),
        in_specs=[
            pl.BlockSpec((NC, BLK, DIM), lambda i: (0, i, 0)),
            pl.BlockSpec((BLK, DIM), lambda i: (i, 0)),
            pl.BlockSpec((BLK, 1), lambda i: (i, 0)),
            pl.BlockSpec((1, DIM), lambda i: (0, 0)),
            pl.BlockSpec((DIM, 1), lambda i: (0, 0)),
            pl.BlockSpec((1, 1), lambda i: (0, 0)),
        ],
        out_specs=pl.BlockSpec((BLK, 1), lambda i: (i, 0)),
        out_shape=jax.ShapeDtypeStruct((N_NODES, 1), jnp.float32),
    )(acc, g, dinv, b, wh, bh)


@jax.jit
def kernel(edge_index, emb, W1, b1, W2, b2, Wh, bh):
    src = edge_index[0].astype(jnp.int32)
    dst = edge_index[1].astype(jnp.int32)
    pad = EP - N_EDGES
    src2d = jnp.concatenate(
        [src, jnp.zeros((pad,), jnp.int32)]).reshape(ROWS, 128)
    dst2d = jnp.concatenate(
        [dst, jnp.full((pad,), N_NODES, jnp.int32)]).reshape(ROWS, 128)

    deg2 = _deg_kernel(dst2d)
    dp = deg2.reshape(NC, DEG_WORDS, 1)

    g1, dinv = _tc1(emb, W1, dp)
    acc1 = _scat_kernel(src2d, dst2d, g1).reshape(NC, ACC_ROWS, DIM)
    g2 = _tc2(acc1, g1, dinv, b1.reshape(1, DIM), W2)
    acc2 = _scat_kernel(src2d, dst2d, g2).reshape(NC, ACC_ROWS, DIM)
    out = _tc3(acc2, g2, dinv, b2.reshape(1, DIM), Wh, bh.reshape(1, 1))
    return out.reshape(N_NODES)


# speedup vs baseline: 27.1744x; 1.0839x over previous
import functools

import jax
import jax.numpy as jnp
from jax import lax
from jax.experimental import pallas as pl
from jax.experimental.pallas import tpu as pltpu
from jax.experimental.pallas import tpu_sc as plsc

N_NODES = 100_000
DIM = 32
N_EDGES = 1_600_000

NC = 2
NS = 16
LANES = 16

ROWS = 12_800
EP = ROWS * 128

DEG_WORDS = 100_352
DEG_TILE = DEG_WORDS // NS
DEG_ROWS_PER_SC = ROWS // NC
DEG_ROWS_PER_TILE = DEG_ROWS_PER_SC // NS
DEG_CHUNKS = DEG_ROWS_PER_TILE // 8

ACC_ROWS = DEG_WORDS
ACC_TILE = ACC_ROWS // NS
ZCOPIES = ACC_TILE // 64
CR = 2
ROWS_SC0 = 8704
ROWS_SC1 = ROWS - ROWS_SC0
RPT_SC0 = ROWS_SC0 // NS
RPT_SC1 = ROWS_SC1 // NS


def _deg_body(dst2d, deg_out, zbuf, ones_v, stage, deg_sh):
    cid = lax.axis_index("c")
    tid = lax.axis_index("s")
    zeros16 = jnp.zeros((LANES,), jnp.float32)
    ones16 = jnp.ones((LANES,), jnp.float32)

    def zb(i, _):
        zbuf[pl.ds(i * LANES, LANES)] = zeros16
        return 0
    lax.fori_loop(0, DEG_TILE // LANES, zb, 0)
    for i in range(128 // LANES):
        ones_v[pl.ds(i * LANES, LANES)] = ones16

    pltpu.sync_copy(zbuf, deg_sh.at[pl.ds(tid * DEG_TILE, DEG_TILE)])
    plsc.subcore_barrier()

    def chunk(m, _):
        r0 = cid * DEG_ROWS_PER_SC + tid * DEG_ROWS_PER_TILE + m * 8
        pltpu.sync_copy(dst2d.at[pl.ds(r0, 8)], stage)
        for j in range(8):
            pltpu.sync_copy(ones_v, deg_sh.at[stage.at[j]], add=True)
        return 0
    lax.fori_loop(0, DEG_CHUNKS, chunk, 0)

    plsc.subcore_barrier()
    pltpu.sync_copy(deg_sh.at[pl.ds(tid * DEG_TILE, DEG_TILE)],
                    deg_out.at[pl.ds(cid * DEG_WORDS + tid * DEG_TILE, DEG_TILE)])


_deg_kernel = pl.kernel(
    _deg_body,
    out_type=jax.ShapeDtypeStruct((NC * DEG_WORDS,), jnp.float32),
    mesh=plsc.VectorSubcoreMesh(
        core_axis_name="c", subcore_axis_name="s", num_cores=NC, num_subcores=NS),
    scratch_types=[
        pltpu.VMEM((DEG_TILE,), jnp.float32),
        pltpu.VMEM((128,), jnp.float32),
        pltpu.VMEM((8, 128), jnp.int32),
        pltpu.VMEM_SHARED((DEG_WORDS,), jnp.float32),
    ],
)


def _scat_body(src2d, dst2d, g, acc_out, zrows, stage_s, stage_d, ldst, rows,
               acc_sh, sem_i, sem_g, sem_s0, sem_s1):
    cid = lax.axis_index("c")
    tid = lax.axis_index("s")
    zeros32 = jnp.zeros((2 * LANES,), jnp.bfloat16)

    def zr(r, _):
        zrows[r, pl.ds(0, 2 * LANES)] = zeros32
        return 0
    lax.fori_loop(0, 64, zr, 0)

    def zc(m, _):
        pltpu.sync_copy(zrows, acc_sh.at[pl.ds(tid * ACC_TILE + m * 64, 64)])
        return 0
    lax.fori_loop(0, ZCOPIES, zc, 0)
    plsc.subcore_barrier()

    rpt = jnp.where(cid == 0, RPT_SC0, RPT_SC1)
    npair = rpt // (2 * CR)
    t0 = jnp.where(cid == 0, 0, ROWS_SC0) + tid * rpt

    def stage_idx(r0, p):
        pltpu.async_copy(src2d.at[pl.ds(r0, CR)], stage_s.at[p], sem_i)
        pltpu.async_copy(dst2d.at[pl.ds(r0, CR)], stage_d.at[p], sem_i)

    def wait_idx(r0, p):
        pltpu.make_async_copy(src2d.at[pl.ds(r0, CR)], stage_s.at[p], sem_i).wait()
        pltpu.make_async_copy(dst2d.at[pl.ds(r0, CR)], stage_d.at[p], sem_i).wait()

    def drain_scat(p, sem):
        for j in range(CR):
            pltpu.make_async_copy(rows.at[p, pl.ds(j * 128, 128)],
                                  acc_sh.at[ldst.at[p, j]], sem).wait()

    def do_chunk(r0, p, sem):
        for j in range(CR):
            pltpu.async_copy(g.at[stage_s.at[p, j]],
                             rows.at[p, pl.ds(j * 128, 128)], sem_g)
        for j in range(CR):
            for k in range(128 // LANES):
                ldst[p, j, pl.ds(k * LANES, LANES)] = (
                    stage_d[p, j, pl.ds(k * LANES, LANES)])
        for j in range(CR):
            pltpu.make_async_copy(g.at[stage_s.at[p, j]],
                                  rows.at[p, pl.ds(j * 128, 128)], sem_g).wait()
        for j in range(CR):
            pltpu.async_copy(rows.at[p, pl.ds(j * 128, 128)],
                             acc_sh.at[ldst.at[p, j]], sem, add=True)

    stage_idx(t0, 0)

    def pair(n, _):
        r0 = t0 + n * 2 * CR
        r1 = r0 + CR
        wait_idx(r0, 0)
        stage_idx(r1, 1)

        @pl.when(n >= 1)
        def _():
            drain_scat(0, sem_s0)
        do_chunk(r0, 0, sem_s0)

        wait_idx(r1, 1)

        @pl.when(n < npair - 1)
        def _():
            stage_idx(r1 + CR, 0)

        @pl.when(n >= 1)
        def _():
            drain_scat(1, sem_s1)
        do_chunk(r1, 1, sem_s1)
        return 0
    lax.fori_loop(0, npair, pair, 0)

    drain_scat(0, sem_s0)
    drain_scat(1, sem_s1)
    plsc.subcore_barrier()
    pltpu.sync_copy(acc_sh.at[pl.ds(tid * ACC_TILE, ACC_TILE)],
                    acc_out.at[pl.ds(cid * ACC_ROWS + tid * ACC_TILE, ACC_TILE)])


_scat_kernel = pl.kernel(
    _scat_body,
    out_type=jax.ShapeDtypeStruct((NC * ACC_ROWS, DIM), jnp.bfloat16),
    mesh=plsc.VectorSubcoreMesh(
        core_axis_name="c", subcore_axis_name="s", num_cores=NC, num_subcores=NS),
    compiler_params=pltpu.CompilerParams(use_tc_tiling_on_sc=False),
    scratch_types=[
        pltpu.VMEM((64, DIM), jnp.bfloat16),
        pltpu.VMEM((2, CR, 128), jnp.int32),
        pltpu.VMEM((2, CR, 128), jnp.int32),
        pltpu.VMEM((2, CR, 128), jnp.int32),
        pltpu.VMEM((2, CR * 128, DIM), jnp.bfloat16),
        pltpu.VMEM_SHARED((ACC_ROWS, DIM), jnp.bfloat16),
        pltpu.SemaphoreType.DMA,
        pltpu.SemaphoreType.DMA,
        pltpu.SemaphoreType.DMA,
        pltpu.SemaphoreType.DMA,
    ],
)


BLK = 2000
GRID = N_NODES // BLK


def _tc1_body(emb_ref, w1_ref, dp_ref, g1_ref, dinv_ref):
    d = dp_ref[...]
    dv = lax.rsqrt(d[0] + d[1] + 1.0)
    h = jnp.dot(emb_ref[...], w1_ref[...], preferred_element_type=jnp.float32)
    g1_ref[...] = (h * dv).astype(jnp.bfloat16)
    dinv_ref[...] = dv


def _tc1(emb, w1, dp):
    return pl.pallas_call(
        _tc1_body,
        grid=(GRID,),
        in_specs=[
            pl.BlockSpec((BLK, DIM), lambda i: (i, 0)),
            pl.BlockSpec((DIM, DIM), lambda i: (0, 0)),
            pl.BlockSpec((NC, BLK, 1), lambda i: (0, i, 0)),
        ],
        out_specs=[
            pl.BlockSpec((BLK, DIM), lambda i: (i, 0)),
            pl.BlockSpec((BLK, 1), lambda i: (i, 0)),
        ],
        out_shape=[
            jax.ShapeDtypeStruct((N_NODES, DIM), jnp.bfloat16),
            jax.ShapeDtypeStruct((N_NODES, 1), jnp.float32),
        ],
    )(emb, w1, dp)


def _tc2_body(acc_ref, g_ref, dinv_ref, b_ref, w_ref, g2_ref):
    dv = dinv_ref[...]
    a = acc_ref[...]
    s = (a[0].astype(jnp.float32) + a[1].astype(jnp.float32)
         + g_ref[...].astype(jnp.float32))
    h = jax.nn.relu(dv * s + b_ref[...])
    g2 = jnp.dot(h, w_ref[...], preferred_element_type=jnp.float32) * dv
    g2_ref[...] = g2.astype(jnp.bfloat16)


def _tc2(acc, g, dinv, b, w):
    return pl.pallas_call(
        _tc2_body,
        grid=(GRID,),
        in_specs=[
            pl.BlockSpec((NC, BLK, DIM), lambda i: (0, i, 0)),
            pl.BlockSpec((BLK, DIM), lambda i: (i, 0)),
            pl.BlockSpec((BLK, 1), lambda i: (i, 0)),
            pl.BlockSpec((1, DIM), lambda i: (0, 0)),
            pl.BlockSpec((DIM, DIM), lambda i: (0, 0)),
        ],
        out_specs=pl.BlockSpec((BLK, DIM), lambda i: (i, 0)),
        out_shape=jax.ShapeDtypeStruct((N_NODES, DIM), jnp.bfloat16),
    )(acc, g, dinv, b, w)


def _tc3_body(acc_ref, g_ref, dinv_ref, b_ref, wh_ref, bh_ref, out_ref):
    dv = dinv_ref[...]
    a = acc_ref[...]
    s = (a[0].astype(jnp.float32) + a[1].astype(jnp.float32)
         + g_ref[...].astype(jnp.float32))
    h = jax.nn.relu(dv * s + b_ref[...])
    z = jnp.dot(h, wh_ref[...], preferred_element_type=jnp.float32) + bh_ref[...]
    out_ref[...] = jax.nn.sigmoid(z)


def _tc3(acc, g, dinv, b, wh, bh):
    return pl.pallas_call(
        _tc3_body,
        grid=(GRID,),
        in_specs=[
            pl.BlockSpec((NC, BLK, DIM), lambda i: (0, i, 0)),
            pl.BlockSpec((BLK, DIM), lambda i: (i, 0)),
            pl.BlockSpec((BLK, 1), lambda i: (i, 0)),
            pl.BlockSpec((1, DIM), lambda i: (0, 0)),
            pl.BlockSpec((DIM, 1), lambda i: (0, 0)),
            pl.BlockSpec((1, 1), lambda i: (0, 0)),
        ],
        out_specs=pl.BlockSpec((BLK, 1), lambda i: (i, 0)),
        out_shape=jax.ShapeDtypeStruct((N_NODES, 1), jnp.float32),
    )(acc, g, dinv, b, wh, bh)


@jax.jit
def kernel(edge_index, emb, W1, b1, W2, b2, Wh, bh):
    src = edge_index[0].astype(jnp.int32)
    dst = edge_index[1].astype(jnp.int32)
    pad = EP - N_EDGES
    src2d = jnp.concatenate(
        [src, jnp.zeros((pad,), jnp.int32)]).reshape(ROWS, 128)
    dst2d = jnp.concatenate(
        [dst, jnp.full((pad,), N_NODES, jnp.int32)]).reshape(ROWS, 128)

    deg2 = _deg_kernel(dst2d)
    dp = deg2.reshape(NC, DEG_WORDS, 1)

    g1, dinv = _tc1(emb, W1, dp)
    acc1 = _scat_kernel(src2d, dst2d, g1).reshape(NC, ACC_ROWS, DIM)
    g2 = _tc2(acc1, g1, dinv, b1.reshape(1, DIM), W2)
    acc2 = _scat_kernel(src2d, dst2d, g2).reshape(NC, ACC_ROWS, DIM)
    out = _tc3(acc2, g2, dinv, b2.reshape(1, DIM), Wh, bh.reshape(1, 1))
    return out.reshape(N_NODES)


# speedup vs baseline: 34.0707x; 1.2538x over previous
import functools

import jax
import jax.numpy as jnp
from jax import lax
from jax.experimental import pallas as pl
from jax.experimental.pallas import tpu as pltpu
from jax.experimental.pallas import tpu_sc as plsc

N_NODES = 100_000
DIM = 32
N_EDGES = 1_600_000

NC = 2
NS = 16
LANES = 16

ROWS = 12_800
EP = ROWS * 128

DEG_WORDS = 100_352
DEG_TILE = DEG_WORDS // NS
DEG_ROWS_PER_SC = ROWS // NC
DEG_ROWS_PER_TILE = DEG_ROWS_PER_SC // NS
DEG_CHUNKS = DEG_ROWS_PER_TILE // 8

ACC_ROWS = DEG_WORDS
ACC_TILE = ACC_ROWS // NS
ZCOPIES = ACC_TILE // 64
CR = 2
ROWS_SC0 = 8704
ROWS_SC1 = ROWS - ROWS_SC0
RPT_SC0 = ROWS_SC0 // NS
RPT_SC1 = ROWS_SC1 // NS


def _deg_body(dst2d, deg_out, zbuf, ones_v, stage, deg_sh):
    cid = lax.axis_index("c")
    tid = lax.axis_index("s")
    zeros16 = jnp.zeros((LANES,), jnp.float32)
    ones16 = jnp.ones((LANES,), jnp.float32)

    def zb(i, _):
        zbuf[pl.ds(i * LANES, LANES)] = zeros16
        return 0
    lax.fori_loop(0, DEG_TILE // LANES, zb, 0)
    for i in range(128 // LANES):
        ones_v[pl.ds(i * LANES, LANES)] = ones16

    pltpu.sync_copy(zbuf, deg_sh.at[pl.ds(tid * DEG_TILE, DEG_TILE)])
    plsc.subcore_barrier()

    def chunk(m, _):
        r0 = cid * DEG_ROWS_PER_SC + tid * DEG_ROWS_PER_TILE + m * 8
        pltpu.sync_copy(dst2d.at[pl.ds(r0, 8)], stage)
        for j in range(8):
            pltpu.sync_copy(ones_v, deg_sh.at[stage.at[j]], add=True)
        return 0
    lax.fori_loop(0, DEG_CHUNKS, chunk, 0)

    plsc.subcore_barrier()
    pltpu.sync_copy(deg_sh.at[pl.ds(tid * DEG_TILE, DEG_TILE)],
                    deg_out.at[pl.ds(cid * DEG_WORDS + tid * DEG_TILE, DEG_TILE)])


_deg_kernel = pl.kernel(
    _deg_body,
    out_type=jax.ShapeDtypeStruct((NC * DEG_WORDS,), jnp.float32),
    mesh=plsc.VectorSubcoreMesh(
        core_axis_name="c", subcore_axis_name="s", num_cores=NC, num_subcores=NS),
    scratch_types=[
        pltpu.VMEM((DEG_TILE,), jnp.float32),
        pltpu.VMEM((128,), jnp.float32),
        pltpu.VMEM((8, 128), jnp.int32),
        pltpu.VMEM_SHARED((DEG_WORDS,), jnp.float32),
    ],
)


def _scat_body(src2d, dst2d, g, acc_out, zrows, stage_s, stage_d, ldst, rows,
               acc_sh, sem_i, sem_g, sem_s0, sem_s1):
    cid = lax.axis_index("c")
    tid = lax.axis_index("s")
    zeros32 = jnp.zeros((2 * LANES,), jnp.bfloat16)

    def zr(r, _):
        zrows[r, pl.ds(0, 2 * LANES)] = zeros32
        return 0
    lax.fori_loop(0, 64, zr, 0)

    def zc(m, _):
        pltpu.sync_copy(zrows, acc_sh.at[pl.ds(tid * ACC_TILE + m * 64, 64)])
        return 0
    lax.fori_loop(0, ZCOPIES, zc, 0)
    plsc.subcore_barrier()

    rpt = jnp.where(cid == 0, RPT_SC0, RPT_SC1)
    npair = rpt // (2 * CR)
    t0 = jnp.where(cid == 0, 0, ROWS_SC0) + tid * rpt

    def stage_idx(r0, p):
        pltpu.async_copy(src2d.at[pl.ds(r0, CR)], stage_s.at[p], sem_i)
        pltpu.async_copy(dst2d.at[pl.ds(r0, CR)], stage_d.at[p], sem_i)

    def wait_idx(r0, p):
        pltpu.make_async_copy(src2d.at[pl.ds(r0, CR)], stage_s.at[p], sem_i).wait()
        pltpu.make_async_copy(dst2d.at[pl.ds(r0, CR)], stage_d.at[p], sem_i).wait()

    def drain_scat(p, sem):
        for j in range(CR):
            pltpu.make_async_copy(rows.at[p, pl.ds(j * 128, 128)],
                                  acc_sh.at[ldst.at[p, j]], sem).wait()

    def do_chunk(r0, p, sem):
        for j in range(CR):
            pltpu.async_copy(g.at[stage_s.at[p, j]],
                             rows.at[p, pl.ds(j * 128, 128)], sem_g)
        for j in range(CR):
            for k in range(128 // LANES):
                ldst[p, j, pl.ds(k * LANES, LANES)] = (
                    stage_d[p, j, pl.ds(k * LANES, LANES)])
        for j in range(CR):
            pltpu.make_async_copy(g.at[stage_s.at[p, j]],
                                  rows.at[p, pl.ds(j * 128, 128)], sem_g).wait()
        for j in range(CR):
            pltpu.async_copy(rows.at[p, pl.ds(j * 128, 128)],
                             acc_sh.at[ldst.at[p, j]], sem, add=True)

    stage_idx(t0, 0)

    def pair(n, _):
        r0 = t0 + n * 2 * CR
        r1 = r0 + CR
        wait_idx(r0, 0)
        stage_idx(r1, 1)

        @pl.when(n >= 1)
        def _():
            drain_scat(0, sem_s0)
        do_chunk(r0, 0, sem_s0)

        wait_idx(r1, 1)

        @pl.when(n < npair - 1)
        def _():
            stage_idx(r1 + CR, 0)

        @pl.when(n >= 1)
        def _():
            drain_scat(1, sem_s1)
        do_chunk(r1, 1, sem_s1)
        return 0
    lax.fori_loop(0, npair, pair, 0)

    drain_scat(0, sem_s0)
    drain_scat(1, sem_s1)
    plsc.subcore_barrier()
    pltpu.sync_copy(acc_sh.at[pl.ds(tid * ACC_TILE, ACC_TILE)],
                    acc_out.at[pl.ds(cid * ACC_ROWS + tid * ACC_TILE, ACC_TILE)])


_scat_kernel = pl.kernel(
    _scat_body,
    out_type=jax.ShapeDtypeStruct((NC * ACC_ROWS, DIM), jnp.bfloat16),
    mesh=plsc.VectorSubcoreMesh(
        core_axis_name="c", subcore_axis_name="s", num_cores=NC, num_subcores=NS),
    compiler_params=pltpu.CompilerParams(use_tc_tiling_on_sc=False),
    scratch_types=[
        pltpu.VMEM((64, DIM), jnp.bfloat16),
        pltpu.VMEM((2, CR, 128), jnp.int32),
        pltpu.VMEM((2, CR, 128), jnp.int32),
        pltpu.VMEM((2, CR, 128), jnp.int32),
        pltpu.VMEM((2, CR * 128, DIM), jnp.bfloat16),
        pltpu.VMEM_SHARED((ACC_ROWS, DIM), jnp.bfloat16),
        pltpu.SemaphoreType.DMA,
        pltpu.SemaphoreType.DMA,
        pltpu.SemaphoreType.DMA,
        pltpu.SemaphoreType.DMA,
    ],
)


BLK = 2048
GRID = 49
BW = BLK // 128


def _scale_rows(h, dvt):
    return jnp.concatenate(
        [h[128 * s:128 * (s + 1)] * dvt[:, s:s + 1] for s in range(BW)], axis=0)


def _tc1_body(emb_ref, w1_ref, dp_ref, g1_ref, dinv_ref):
    d = dp_ref[...]
    dv = lax.rsqrt(d[0] + d[1] + 1.0)
    dinv_ref[...] = dv
    dvt = dv.T
    h = jnp.dot(emb_ref[...], w1_ref[...], preferred_element_type=jnp.float32)
    g1_ref[...] = _scale_rows(h, dvt).astype(jnp.bfloat16)


def _tc1(emb, w1, dp):
    return pl.pallas_call(
        _tc1_body,
        grid=(GRID,),
        in_specs=[
            pl.BlockSpec((BLK, DIM), lambda i: (i, 0)),
            pl.BlockSpec((DIM, DIM), lambda i: (0, 0)),
            pl.BlockSpec((NC, BW, 128), lambda i: (0, i, 0)),
        ],
        out_specs=[
            pl.BlockSpec((BLK, DIM), lambda i: (i, 0)),
            pl.BlockSpec((BW, 128), lambda i: (i, 0)),
        ],
        out_shape=[
            jax.ShapeDtypeStruct((N_NODES, DIM), jnp.bfloat16),
            jax.ShapeDtypeStruct((DEG_WORDS // 128, 128), jnp.float32),
        ],
    )(emb, w1, dp)


def _tc2_body(acc_ref, g_ref, dinv_ref, b_ref, w_ref, g2_ref):
    dvt = dinv_ref[...].T
    a = acc_ref[...]
    s = (a[0].astype(jnp.float32) + a[1].astype(jnp.float32)
         + g_ref[...].astype(jnp.float32))
    h = jax.nn.relu(_scale_rows(s, dvt) + b_ref[...])
    g2 = jnp.dot(h, w_ref[...], preferred_element_type=jnp.float32)
    g2_ref[...] = _scale_rows(g2, dvt).astype(jnp.bfloat16)


def _tc2(acc, g, dinv, b, w):
    return pl.pallas_call(
        _tc2_body,
        grid=(GRID,),
        in_specs=[
            pl.BlockSpec((NC, BLK, DIM), lambda i: (0, i, 0)),
            pl.BlockSpec((BLK, DIM), lambda i: (i, 0)),
            pl.BlockSpec((BW, 128), lambda i: (i, 0)),
            pl.BlockSpec((1, DIM), lambda i: (0, 0)),
            pl.BlockSpec((DIM, DIM), lambda i: (0, 0)),
        ],
        out_specs=pl.BlockSpec((BLK, DIM), lambda i: (i, 0)),
        out_shape=jax.ShapeDtypeStruct((N_NODES, DIM), jnp.bfloat16),
    )(acc, g, dinv, b, w)


def _tc3_body(acc_ref, g_ref, dinv_ref, b_ref, wh_ref, bh_ref, out_ref):
    dvt = dinv_ref[...].T
    a = acc_ref[...]
    s = (a[0].astype(jnp.float32) + a[1].astype(jnp.float32)
         + g_ref[...].astype(jnp.float32))
    h = jax.nn.relu(_scale_rows(s, dvt) + b_ref[...])
    z = jnp.dot(h, wh_ref[...], preferred_element_type=jnp.float32) + bh_ref[...]
    sg = jax.nn.sigmoid(z)
    out_ref[...] = jnp.concatenate(
        [sg[128 * s:128 * (s + 1)].T for s in range(BW)], axis=0)


def _tc3(acc, g, dinv, b, wh, bh):
    return pl.pallas_call(
        _tc3_body,
        grid=(GRID,),
        in_specs=[
            pl.BlockSpec((NC, BLK, DIM), lambda i: (0, i, 0)),
            pl.BlockSpec((BLK, DIM), lambda i: (i, 0)),
            pl.BlockSpec((BW, 128), lambda i: (i, 0)),
            pl.BlockSpec((1, DIM), lambda i: (0, 0)),
            pl.BlockSpec((DIM, 1), lambda i: (0, 0)),
            pl.BlockSpec((1, 1), lambda i: (0, 0)),
        ],
        out_specs=pl.BlockSpec((BW, 128), lambda i: (i, 0)),
        out_shape=jax.ShapeDtypeStruct((DEG_WORDS // 128, 128), jnp.float32),
    )(acc, g, dinv, b, wh, bh)


@jax.jit
def kernel(edge_index, emb, W1, b1, W2, b2, Wh, bh):
    src = edge_index[0].astype(jnp.int32)
    dst = edge_index[1].astype(jnp.int32)
    pad = EP - N_EDGES
    src2d = jnp.concatenate(
        [src, jnp.zeros((pad,), jnp.int32)]).reshape(ROWS, 128)
    dst2d = jnp.concatenate(
        [dst, jnp.full((pad,), N_NODES, jnp.int32)]).reshape(ROWS, 128)

    deg2 = _deg_kernel(dst2d)
    dp = deg2.reshape(NC, DEG_WORDS // 128, 128)

    g1, dinv = _tc1(emb, W1, dp)
    acc1 = _scat_kernel(src2d, dst2d, g1).reshape(NC, ACC_ROWS, DIM)
    g2 = _tc2(acc1, g1, dinv, b1.reshape(1, DIM), W2)
    acc2 = _scat_kernel(src2d, dst2d, g2).reshape(NC, ACC_ROWS, DIM)
    out = _tc3(acc2, g2, dinv, b2.reshape(1, DIM), Wh, bh.reshape(1, 1))
    return out.reshape(DEG_WORDS)[:N_NODES]


# speedup vs baseline: 36.2505x; 1.0640x over previous
import functools

import jax
import jax.numpy as jnp
from jax import lax
from jax.experimental import pallas as pl
from jax.experimental.pallas import tpu as pltpu
from jax.experimental.pallas import tpu_sc as plsc

N_NODES = 100_000
DIM = 32
N_EDGES = 1_600_000

NC = 2
NS = 16
LANES = 16

ROWS = N_EDGES // 128
PAIRS = ROWS // 4

DEG_WORDS = 100_352
DEG_TILE = DEG_WORDS // NS
DEG_SC0 = 1568
DEG_SC1 = PAIRS - DEG_SC0

ACC_ROWS = DEG_WORDS
ACC_TILE = ACC_ROWS // NS
ZCOPIES = ACC_TILE // 64
CR = 2
PAIRS_SC0 = 2125
PAIRS_SC1 = PAIRS - PAIRS_SC0


def _deg_body(dst2d, deg_out, zbuf, ones_v, stage, deg_sh):
    cid = lax.axis_index("c")
    tid = lax.axis_index("s")
    zeros16 = jnp.zeros((LANES,), jnp.float32)
    ones16 = jnp.ones((LANES,), jnp.float32)

    def zb(i, _):
        zbuf[pl.ds(i * LANES, LANES)] = zeros16
        return 0
    lax.fori_loop(0, DEG_TILE // LANES, zb, 0)
    for i in range(128 // LANES):
        ones_v[pl.ds(i * LANES, LANES)] = ones16

    pltpu.sync_copy(zbuf, deg_sh.at[pl.ds(tid * DEG_TILE, DEG_TILE)])
    plsc.subcore_barrier()

    n_ch = jnp.where(cid == 0, 98, jnp.where(tid < 5, 98, 97))
    start = jnp.where(cid == 0, 98 * tid,
                      DEG_SC0 + 97 * tid + jnp.minimum(tid, 5))

    def chunk(m, _):
        r0 = (start + m) * 4
        pltpu.sync_copy(dst2d.at[pl.ds(r0, 4)], stage)
        for j in range(4):
            pltpu.sync_copy(ones_v, deg_sh.at[stage.at[j]], add=True)
        return 0
    lax.fori_loop(0, n_ch, chunk, 0)

    plsc.subcore_barrier()
    pltpu.sync_copy(deg_sh.at[pl.ds(tid * DEG_TILE, DEG_TILE)],
                    deg_out.at[pl.ds(cid * DEG_WORDS + tid * DEG_TILE, DEG_TILE)])


_deg_kernel = pl.kernel(
    _deg_body,
    out_type=jax.ShapeDtypeStruct((NC * DEG_WORDS,), jnp.float32),
    mesh=plsc.VectorSubcoreMesh(
        core_axis_name="c", subcore_axis_name="s", num_cores=NC, num_subcores=NS),
    scratch_types=[
        pltpu.VMEM((DEG_TILE,), jnp.float32),
        pltpu.VMEM((128,), jnp.float32),
        pltpu.VMEM((4, 128), jnp.int32),
        pltpu.VMEM_SHARED((DEG_WORDS,), jnp.float32),
    ],
)


def _scat_body(src2d, dst2d, g, acc_out, zrows, stage_s, stage_d, ldst, rows,
               acc_sh, sem_i, sem_g, sem_s0, sem_s1):
    cid = lax.axis_index("c")
    tid = lax.axis_index("s")
    zeros32 = jnp.zeros((2 * LANES,), jnp.bfloat16)

    def zr(r, _):
        zrows[r, pl.ds(0, 2 * LANES)] = zeros32
        return 0
    lax.fori_loop(0, 64, zr, 0)

    def zc(m, _):
        pltpu.sync_copy(zrows, acc_sh.at[pl.ds(tid * ACC_TILE + m * 64, 64)])
        return 0
    lax.fori_loop(0, ZCOPIES, zc, 0)
    plsc.subcore_barrier()

    npair = jnp.where(cid == 0,
                      jnp.where(tid < 13, 133, 132),
                      jnp.where(tid < 8, 63, 62))
    start_p = jnp.where(cid == 0,
                        132 * tid + jnp.minimum(tid, 13),
                        PAIRS_SC0 + 62 * tid + jnp.minimum(tid, 8))
    t0 = start_p * 2 * CR

    def stage_idx(r0, p):
        pltpu.async_copy(src2d.at[pl.ds(r0, CR)], stage_s.at[p], sem_i)
        pltpu.async_copy(dst2d.at[pl.ds(r0, CR)], stage_d.at[p], sem_i)

    def wait_idx(r0, p):
        pltpu.make_async_copy(src2d.at[pl.ds(r0, CR)], stage_s.at[p], sem_i).wait()
        pltpu.make_async_copy(dst2d.at[pl.ds(r0, CR)], stage_d.at[p], sem_i).wait()

    def drain_scat(p, sem):
        for j in range(CR):
            pltpu.make_async_copy(rows.at[p, pl.ds(j * 128, 128)],
                                  acc_sh.at[ldst.at[p, j]], sem).wait()

    def do_chunk(r0, p, sem):
        for j in range(CR):
            pltpu.async_copy(g.at[stage_s.at[p, j]],
                             rows.at[p, pl.ds(j * 128, 128)], sem_g)
        for j in range(CR):
            for k in range(128 // LANES):
                ldst[p, j, pl.ds(k * LANES, LANES)] = (
                    stage_d[p, j, pl.ds(k * LANES, LANES)])
        for j in range(CR):
            pltpu.make_async_copy(g.at[stage_s.at[p, j]],
                                  rows.at[p, pl.ds(j * 128, 128)], sem_g).wait()
        for j in range(CR):
            pltpu.async_copy(rows.at[p, pl.ds(j * 128, 128)],
                             acc_sh.at[ldst.at[p, j]], sem, add=True)

    stage_idx(t0, 0)

    def pair(n, _):
        r0 = t0 + n * 2 * CR
        r1 = r0 + CR
        wait_idx(r0, 0)
        stage_idx(r1, 1)

        @pl.when(n >= 1)
        def _():
            drain_scat(0, sem_s0)
        do_chunk(r0, 0, sem_s0)

        wait_idx(r1, 1)

        @pl.when(n < npair - 1)
        def _():
            stage_idx(r1 + CR, 0)

        @pl.when(n >= 1)
        def _():
            drain_scat(1, sem_s1)
        do_chunk(r1, 1, sem_s1)
        return 0
    lax.fori_loop(0, npair, pair, 0)

    drain_scat(0, sem_s0)
    drain_scat(1, sem_s1)
    plsc.subcore_barrier()
    pltpu.sync_copy(acc_sh.at[pl.ds(tid * ACC_TILE, ACC_TILE)],
                    acc_out.at[pl.ds(cid * ACC_ROWS + tid * ACC_TILE, ACC_TILE)])


_scat_kernel = pl.kernel(
    _scat_body,
    out_type=jax.ShapeDtypeStruct((NC * ACC_ROWS, DIM), jnp.bfloat16),
    mesh=plsc.VectorSubcoreMesh(
        core_axis_name="c", subcore_axis_name="s", num_cores=NC, num_subcores=NS),
    compiler_params=pltpu.CompilerParams(use_tc_tiling_on_sc=False),
    scratch_types=[
        pltpu.VMEM((64, DIM), jnp.bfloat16),
        pltpu.VMEM((2, CR, 128), jnp.int32),
        pltpu.VMEM((2, CR, 128), jnp.int32),
        pltpu.VMEM((2, CR, 128), jnp.int32),
        pltpu.VMEM((2, CR * 128, DIM), jnp.bfloat16),
        pltpu.VMEM_SHARED((ACC_ROWS, DIM), jnp.bfloat16),
        pltpu.SemaphoreType.DMA,
        pltpu.SemaphoreType.DMA,
        pltpu.SemaphoreType.DMA,
        pltpu.SemaphoreType.DMA,
    ],
)


BLK = 2048
GRID = 49
BW = BLK // 128


def _scale_rows(h, dvt):
    return jnp.concatenate(
        [h[128 * s:128 * (s + 1)] * dvt[:, s:s + 1] for s in range(BW)], axis=0)


def _tc1_body(emb_ref, w1_ref, dp_ref, g1_ref, dinv_ref):
    d = dp_ref[...]
    dv = lax.rsqrt(d[0] + d[1] + 1.0)
    dinv_ref[...] = dv
    dvt = dv.T
    h = jnp.dot(emb_ref[...], w1_ref[...], preferred_element_type=jnp.float32)
    g1_ref[...] = _scale_rows(h, dvt).astype(jnp.bfloat16)


def _tc1(emb, w1, dp):
    return pl.pallas_call(
        _tc1_body,
        grid=(GRID,),
        in_specs=[
            pl.BlockSpec((BLK, DIM), lambda i: (i, 0)),
            pl.BlockSpec((DIM, DIM), lambda i: (0, 0)),
            pl.BlockSpec((NC, BW, 128), lambda i: (0, i, 0)),
        ],
        out_specs=[
            pl.BlockSpec((BLK, DIM), lambda i: (i, 0)),
            pl.BlockSpec((BW, 128), lambda i: (i, 0)),
        ],
        out_shape=[
            jax.ShapeDtypeStruct((N_NODES, DIM), jnp.bfloat16),
            jax.ShapeDtypeStruct((DEG_WORDS // 128, 128), jnp.float32),
        ],
    )(emb, w1, dp)


def _tc2_body(acc_ref, g_ref, dinv_ref, b_ref, w_ref, g2_ref):
    dvt = dinv_ref[...].T
    a = acc_ref[...]
    s = (a[0].astype(jnp.float32) + a[1].astype(jnp.float32)
         + g_ref[...].astype(jnp.float32))
    h = jax.nn.relu(_scale_rows(s, dvt) + b_ref[...])
    g2 = jnp.dot(h, w_ref[...], preferred_element_type=jnp.float32)
    g2_ref[...] = _scale_rows(g2, dvt).astype(jnp.bfloat16)


def _tc2(acc, g, dinv, b, w):
    return pl.pallas_call(
        _tc2_body,
        grid=(GRID,),
        in_specs=[
            pl.BlockSpec((NC, BLK, DIM), lambda i: (0, i, 0)),
            pl.BlockSpec((BLK, DIM), lambda i: (i, 0)),
            pl.BlockSpec((BW, 128), lambda i: (i, 0)),
            pl.BlockSpec((1, DIM), lambda i: (0, 0)),
            pl.BlockSpec((DIM, DIM), lambda i: (0, 0)),
        ],
        out_specs=pl.BlockSpec((BLK, DIM), lambda i: (i, 0)),
        out_shape=jax.ShapeDtypeStruct((N_NODES, DIM), jnp.bfloat16),
    )(acc, g, dinv, b, w)


def _tc3_body(acc_ref, g_ref, dinv_ref, b_ref, wh_ref, bh_ref, out_ref):
    dvt = dinv_ref[...].T
    a = acc_ref[...]
    s = (a[0].astype(jnp.float32) + a[1].astype(jnp.float32)
         + g_ref[...].astype(jnp.float32))
    h = jax.nn.relu(_scale_rows(s, dvt) + b_ref[...])
    z = jnp.dot(h, wh_ref[...], preferred_element_type=jnp.float32) + bh_ref[...]
    sg = jax.nn.sigmoid(z)
    out_ref[...] = jnp.concatenate(
        [sg[128 * s:128 * (s + 1)].T for s in range(BW)], axis=0)


def _tc3(acc, g, dinv, b, wh, bh):
    return pl.pallas_call(
        _tc3_body,
        grid=(GRID,),
        in_specs=[
            pl.BlockSpec((NC, BLK, DIM), lambda i: (0, i, 0)),
            pl.BlockSpec((BLK, DIM), lambda i: (i, 0)),
            pl.BlockSpec((BW, 128), lambda i: (i, 0)),
            pl.BlockSpec((1, DIM), lambda i: (0, 0)),
            pl.BlockSpec((DIM, 1), lambda i: (0, 0)),
            pl.BlockSpec((1, 1), lambda i: (0, 0)),
        ],
        out_specs=pl.BlockSpec((BW, 128), lambda i: (i, 0)),
        out_shape=jax.ShapeDtypeStruct((DEG_WORDS // 128, 128), jnp.float32),
    )(acc, g, dinv, b, wh, bh)


@jax.jit
def kernel(edge_index, emb, W1, b1, W2, b2, Wh, bh):
    src2d = edge_index[0].astype(jnp.int32).reshape(ROWS, 128)
    dst2d = edge_index[1].astype(jnp.int32).reshape(ROWS, 128)

    deg2 = _deg_kernel(dst2d)
    dp = deg2.reshape(NC, DEG_WORDS // 128, 128)

    g1, dinv = _tc1(emb, W1, dp)
    acc1 = _scat_kernel(src2d, dst2d, g1).reshape(NC, ACC_ROWS, DIM)
    g2 = _tc2(acc1, g1, dinv, b1.reshape(1, DIM), W2)
    acc2 = _scat_kernel(src2d, dst2d, g2).reshape(NC, ACC_ROWS, DIM)
    out = _tc3(acc2, g2, dinv, b2.reshape(1, DIM), Wh, bh.reshape(1, 1))
    return out.reshape(DEG_WORDS)[:N_NODES]
